# baseline (speedup 1.0000x reference)
import functools

import jax
import jax.numpy as jnp
from jax.experimental import pallas as pl
from jax.experimental.pallas import tpu as pltpu

TN = 4096


def _sigmoid(v):
    return 0.5 * jnp.tanh(0.5 * v) + 0.5


def _swish(v):
    return v * _sigmoid(v)


def _onehot_t(i, tn, starts, ends):
    idx = i * tn + jax.lax.broadcasted_iota(jnp.int32, (1, tn), 1)
    return ((idx >= starts) & (idx < ends)).astype(jnp.float32)


def _segdot(oh_t, v):
    return jax.lax.dot_general(
        oh_t, v, (((1,), (0,)), ((), ())),
        preferred_element_type=jnp.float32)


def _body(nblk, inp_ref, w_in_ref, b_in_ref, w_emb_ref, b_emb_ref,
          starts_ref, ends_ref, starts_rep_ref, ends_rep_ref,
          rtile_ref, cnt_ref, den_ref,
          wrs0_ref, wrsea_ref, wrspa_ref, wrses_ref, wrsps_ref,
          brs_ref, weis_ref, weie_ref, bei_ref, wlsei_ref, wlsx_ref,
          bls_ref, wlm_ref, blm_ref, wlo_ref, blo_ref,
          out_ref, esum_acc, gsum_acc, stats_s, acc_ref):
    j = pl.program_id(0)
    i = pl.program_id(1)

    @pl.when((j == 0) & (i == 0))
    def _init():
        esum_acc[...] = jnp.zeros_like(esum_acc)
        gsum_acc[...] = jnp.zeros_like(gsum_acc)

    x = _swish(jnp.dot(inp_ref[...], w_in_ref[...],
                       preferred_element_type=jnp.float32) + b_in_ref[...])
    emb = _swish(jnp.dot(x, w_emb_ref[...],
                         preferred_element_type=jnp.float32) + b_emb_ref[...])
    tn, embd = emb.shape

    @pl.when(j == 0)
    def _accumulate_stats():
        oh = _onehot_t(i, tn, starts_ref[...], ends_ref[...])
        esum_acc[...] += _segdot(oh, emb)
        emb_rep = jnp.dot(emb, rtile_ref[...],
                          preferred_element_type=jnp.float32)
        idxc = i * tn + jax.lax.broadcasted_iota(jnp.int32, (tn, 1), 0)
        mask = (idxc >= starts_rep_ref[...]) & (idxc < ends_rep_ref[...])
        h = jnp.where(mask, emb_rep, 0.0)
        gsum_acc[...] += jax.lax.dot_general(
            h, emb, (((0,), (0,)), ((), ())),
            preferred_element_type=jnp.float32)

    @pl.when((j == 1) & (i == 0))
    def _finalize_stats():
        esum = esum_acc[...]
        b, _ = esum.shape
        den = den_ref[...]
        lin = (cnt_ref[...] * wrs0_ref[...]
               + jnp.dot(esum / den, wrsea_ref[...],
                         preferred_element_type=jnp.float32)
               + jnp.dot(esum, wrses_ref[...],
                         preferred_element_type=jnp.float32)
               + brs_ref[...])
        for a in range(embd):
            blk = gsum_acc[a * b:(a + 1) * b, :]
            wpa = wrspa_ref[a * embd:(a + 1) * embd, :]
            wps = wrsps_ref[a * embd:(a + 1) * embd, :]
            lin += (jnp.dot(blk / den, wpa, preferred_element_type=jnp.float32)
                    + jnp.dot(blk, wps, preferred_element_type=jnp.float32))
        stats_s[...] = jnp.dot(_swish(lin), weis_ref[...],
                               preferred_element_type=jnp.float32)
        acc_ref[...] = jnp.zeros_like(acc_ref)

    @pl.when(j == 1)
    def _token_layers():
        oh = _onehot_t(i, tn, starts_ref[...], ends_ref[...])
        stok = jax.lax.dot_general(oh, stats_s[...], (((0,), (0,)), ((), ())),
                                   preferred_element_type=jnp.float32)
        ei = _swish(stok
                    + jnp.dot(emb, weie_ref[...],
                              preferred_element_type=jnp.float32)
                    + bei_ref[...])
        ls = _swish(jnp.dot(ei, wlsei_ref[...],
                            preferred_element_type=jnp.float32)
                    + jnp.dot(x, wlsx_ref[...],
                              preferred_element_type=jnp.float32)
                    + bls_ref[...])
        mult = _sigmoid(jnp.dot(ei, wlm_ref[...],
                                preferred_element_type=jnp.float32)
                        + blm_ref[...])
        acc_ref[...] += _segdot(oh, ls * mult)

        @pl.when(i == nblk - 1)
        def _final_out():
            totals = acc_ref[...] / den_ref[...]
            out_ref[...] = (jnp.dot(totals, wlo_ref[...],
                                    preferred_element_type=jnp.float32)
                            + blo_ref[...])


def kernel(inputs, splits, W_in, b_in, W_emb, b_emb, W_rs, b_rs,
           W_ei, b_ei, W_ls, b_ls, W_lm, b_lm, W_lo, b_lo):
    N, D_IN = inputs.shape
    B = splits.shape[0]
    MIX = W_in.shape[1]
    EMB = W_emb.shape[1]
    RS = W_rs.shape[1]
    EI = W_ei.shape[1]
    LS = W_ls.shape[1]
    LO = W_lo.shape[1]
    nblk = N // TN

    sp = splits.astype(jnp.int32)
    starts = sp[:, 0].reshape(B, 1)
    ends = sp[:, 1].reshape(B, 1)
    cnt_col = (sp[:, 1] - sp[:, 0]).astype(jnp.float32).reshape(B, 1)
    den_col = jnp.maximum(cnt_col, 1.0)
    starts_rep = jnp.tile(sp[:, 0], EMB).reshape(1, B * EMB)
    ends_rep = jnp.tile(sp[:, 1], EMB).reshape(1, B * EMB)
    rtile = jnp.repeat(jnp.eye(EMB, dtype=jnp.float32), B, axis=1)

    wrs0 = W_rs[0:1]
    wrsea = W_rs[1:1 + EMB]
    wrspa = W_rs[1 + EMB:1 + EMB + EMB * EMB]
    wrses = W_rs[1 + EMB + EMB * EMB:1 + 2 * EMB + EMB * EMB]
    wrsps = W_rs[1 + 2 * EMB + EMB * EMB:]

    full = lambda shape: pl.BlockSpec(shape, lambda j, i: (0, 0))

    out = pl.pallas_call(
        functools.partial(_body, nblk),
        grid=(2, nblk),
        in_specs=[
            pl.BlockSpec((TN, D_IN), lambda j, i: (i, 0)),
            full((D_IN, MIX)), full((1, MIX)),
            full((MIX, EMB)), full((1, EMB)),
            full((B, 1)), full((B, 1)),
            full((1, B * EMB)), full((1, B * EMB)), full((EMB, B * EMB)),
            full((B, 1)), full((B, 1)),
            full((1, RS)), full((EMB, RS)), full((EMB * EMB, RS)),
            full((EMB, RS)), full((EMB * EMB, RS)), full((1, RS)),
            full((RS, EI)), full((EMB, EI)), full((1, EI)),
            full((EI, LS)), full((MIX, LS)), full((1, LS)),
            full((EI, LS)), full((1, LS)),
            full((LS, LO)), full((1, LO)),
        ],
        out_specs=full((B, LO)),
        out_shape=jax.ShapeDtypeStruct((B, LO), jnp.float32),
        scratch_shapes=[
            pltpu.VMEM((B, EMB), jnp.float32),
            pltpu.VMEM((B * EMB, EMB), jnp.float32),
            pltpu.VMEM((B, EI), jnp.float32),
            pltpu.VMEM((B, LS), jnp.float32),
        ],
    )(inputs, W_in, b_in.reshape(1, MIX), W_emb, b_emb.reshape(1, EMB),
      starts, ends, starts_rep, ends_rep, rtile, cnt_col, den_col,
      wrs0, wrsea, wrspa, wrses, wrsps, b_rs.reshape(1, RS),
      W_ei[:RS], W_ei[RS:], b_ei.reshape(1, EI),
      W_ls[:EI], W_ls[EI:], b_ls.reshape(1, LS),
      W_lm, b_lm.reshape(1, LS), W_lo, b_lo.reshape(1, LO))
    return out


# speedup vs baseline: 1.0706x; 1.0706x over previous
import functools

import jax
import jax.numpy as jnp
from jax.experimental import pallas as pl
from jax.experimental.pallas import tpu as pltpu

TN = 4096


def _sigmoid(v):
    return 0.5 * jnp.tanh(0.5 * v) + 0.5


def _swish(v):
    return v * _sigmoid(v)


def _onehot_t(i, tn, starts, ends):
    idx = i * tn + jax.lax.broadcasted_iota(jnp.int32, (1, tn), 1)
    return ((idx >= starts) & (idx < ends)).astype(jnp.float32)


def _segdot(oh_t, v):
    return jax.lax.dot_general(
        oh_t, v, (((1,), (0,)), ((), ())),
        preferred_element_type=jnp.float32)


def _pass1_body(nblk, inp_ref, w_in_ref, b_in_ref, w_emb_ref, b_emb_ref,
                starts_ref, ends_ref, starts_rep_ref, ends_rep_ref,
                rtile_ref, cnt_ref, den_ref,
                wrs0_ref, wrsea_ref, wrspa_ref, wrses_ref, wrsps_ref,
                brs_ref, weis_ref, x_out_ref, emb_out_ref, stats_ref,
                esum_acc, gsum_acc):
    i = pl.program_id(0)

    @pl.when(i == 0)
    def _init():
        esum_acc[...] = jnp.zeros_like(esum_acc)
        gsum_acc[...] = jnp.zeros_like(gsum_acc)

    x = _swish(jnp.dot(inp_ref[...], w_in_ref[...],
                       preferred_element_type=jnp.float32) + b_in_ref[...])
    emb = _swish(jnp.dot(x, w_emb_ref[...],
                         preferred_element_type=jnp.float32) + b_emb_ref[...])
    x_out_ref[...] = x.astype(jnp.bfloat16)
    emb_out_ref[...] = emb.astype(jnp.bfloat16)

    tn, embd = emb.shape
    oh = _onehot_t(i, tn, starts_ref[...], ends_ref[...])
    esum_acc[...] += _segdot(oh, emb)
    emb_rep = jnp.dot(emb, rtile_ref[...], preferred_element_type=jnp.float32)
    idxc = i * tn + jax.lax.broadcasted_iota(jnp.int32, (tn, 1), 0)
    mask = (idxc >= starts_rep_ref[...]) & (idxc < ends_rep_ref[...])
    h = jnp.where(mask, emb_rep, 0.0)
    gsum_acc[...] += jax.lax.dot_general(
        h, emb, (((0,), (0,)), ((), ())), preferred_element_type=jnp.float32)

    @pl.when(i == nblk - 1)
    def _finalize():
        esum = esum_acc[...]
        b, _ = esum.shape
        den = den_ref[...]
        lin = (cnt_ref[...] * wrs0_ref[...]
               + jnp.dot(esum / den, wrsea_ref[...],
                         preferred_element_type=jnp.float32)
               + jnp.dot(esum, wrses_ref[...],
                         preferred_element_type=jnp.float32)
               + brs_ref[...])
        for a in range(embd):
            blk = gsum_acc[a * b:(a + 1) * b, :]
            wpa = wrspa_ref[a * embd:(a + 1) * embd, :]
            wps = wrsps_ref[a * embd:(a + 1) * embd, :]
            lin += (jnp.dot(blk / den, wpa, preferred_element_type=jnp.float32)
                    + jnp.dot(blk, wps, preferred_element_type=jnp.float32))
        stats_ref[...] = jnp.dot(_swish(lin), weis_ref[...],
                                 preferred_element_type=jnp.float32)


def _pass2_body(nblk, x_ref, emb_ref, stats_ref, starts_ref, ends_ref,
                den_ref, weie_ref, bei_ref, wlsei_ref, wlsx_ref,
                bls_ref, wlm_ref, blm_ref, wlo_ref, blo_ref,
                out_ref, acc_ref):
    i = pl.program_id(0)

    @pl.when(i == 0)
    def _init():
        acc_ref[...] = jnp.zeros_like(acc_ref)

    oh = _onehot_t(i, x_ref.shape[0], starts_ref[...], ends_ref[...])
    stok = jax.lax.dot_general(oh, stats_ref[...], (((0,), (0,)), ((), ())),
                               preferred_element_type=jnp.float32)
    ei = _swish(stok
                + jnp.dot(emb_ref[...], weie_ref[...],
                          preferred_element_type=jnp.float32)
                + bei_ref[...])
    ls = _swish(jnp.dot(ei, wlsei_ref[...],
                        preferred_element_type=jnp.float32)
                + jnp.dot(x_ref[...], wlsx_ref[...],
                          preferred_element_type=jnp.float32)
                + bls_ref[...])
    mult = _sigmoid(jnp.dot(ei, wlm_ref[...],
                            preferred_element_type=jnp.float32)
                    + blm_ref[...])
    acc_ref[...] += _segdot(oh, ls * mult)

    @pl.when(i == nblk - 1)
    def _finalize():
        totals = acc_ref[...] / den_ref[...]
        out_ref[...] = (jnp.dot(totals, wlo_ref[...],
                                preferred_element_type=jnp.float32)
                        + blo_ref[...])


def kernel(inputs, splits, W_in, b_in, W_emb, b_emb, W_rs, b_rs,
           W_ei, b_ei, W_ls, b_ls, W_lm, b_lm, W_lo, b_lo):
    N, D_IN = inputs.shape
    B = splits.shape[0]
    MIX = W_in.shape[1]
    EMB = W_emb.shape[1]
    RS = W_rs.shape[1]
    EI = W_ei.shape[1]
    LS = W_ls.shape[1]
    LO = W_lo.shape[1]
    nblk = N // TN

    sp = splits.astype(jnp.int32)
    starts = sp[:, 0].reshape(B, 1)
    ends = sp[:, 1].reshape(B, 1)
    cnt_col = (sp[:, 1] - sp[:, 0]).astype(jnp.float32).reshape(B, 1)
    den_col = jnp.maximum(cnt_col, 1.0)
    starts_rep = jnp.tile(sp[:, 0], EMB).reshape(1, B * EMB)
    ends_rep = jnp.tile(sp[:, 1], EMB).reshape(1, B * EMB)
    rtile = jnp.repeat(jnp.eye(EMB, dtype=jnp.float32), B, axis=1)

    wrs0 = W_rs[0:1]
    wrsea = W_rs[1:1 + EMB]
    wrspa = W_rs[1 + EMB:1 + EMB + EMB * EMB]
    wrses = W_rs[1 + EMB + EMB * EMB:1 + 2 * EMB + EMB * EMB]
    wrsps = W_rs[1 + 2 * EMB + EMB * EMB:]

    full = lambda shape: pl.BlockSpec(shape, lambda i: (0, 0))

    x, emb, stats_red = pl.pallas_call(
        functools.partial(_pass1_body, nblk),
        grid=(nblk,),
        in_specs=[
            pl.BlockSpec((TN, D_IN), lambda i: (i, 0)),
            full((D_IN, MIX)), full((1, MIX)),
            full((MIX, EMB)), full((1, EMB)),
            full((B, 1)), full((B, 1)),
            full((1, B * EMB)), full((1, B * EMB)), full((EMB, B * EMB)),
            full((B, 1)), full((B, 1)),
            full((1, RS)), full((EMB, RS)), full((EMB * EMB, RS)),
            full((EMB, RS)), full((EMB * EMB, RS)), full((1, RS)),
            full((RS, EI)),
        ],
        out_specs=[
            pl.BlockSpec((TN, MIX), lambda i: (i, 0)),
            pl.BlockSpec((TN, EMB), lambda i: (i, 0)),
            full((B, EI)),
        ],
        out_shape=[
            jax.ShapeDtypeStruct((N, MIX), jnp.bfloat16),
            jax.ShapeDtypeStruct((N, EMB), jnp.bfloat16),
            jax.ShapeDtypeStruct((B, EI), jnp.float32),
        ],
        scratch_shapes=[
            pltpu.VMEM((B, EMB), jnp.float32),
            pltpu.VMEM((B * EMB, EMB), jnp.float32),
        ],
    )(inputs, W_in, b_in.reshape(1, MIX), W_emb, b_emb.reshape(1, EMB),
      starts, ends, starts_rep, ends_rep, rtile, cnt_col, den_col,
      wrs0, wrsea, wrspa, wrses, wrsps, b_rs.reshape(1, RS), W_ei[:RS])

    out = pl.pallas_call(
        functools.partial(_pass2_body, nblk),
        grid=(nblk,),
        in_specs=[
            pl.BlockSpec((TN, MIX), lambda i: (i, 0)),
            pl.BlockSpec((TN, EMB), lambda i: (i, 0)),
            full((B, EI)),
            full((B, 1)), full((B, 1)), full((B, 1)),
            full((EMB, EI)), full((1, EI)),
            full((EI, LS)), full((MIX, LS)), full((1, LS)),
            full((EI, LS)), full((1, LS)),
            full((LS, LO)), full((1, LO)),
        ],
        out_specs=full((B, LO)),
        out_shape=jax.ShapeDtypeStruct((B, LO), jnp.float32),
        scratch_shapes=[pltpu.VMEM((B, LS), jnp.float32)],
    )(x, emb, stats_red, starts, ends, den_col,
      W_ei[RS:], b_ei.reshape(1, EI),
      W_ls[:EI], W_ls[EI:], b_ls.reshape(1, LS),
      W_lm, b_lm.reshape(1, LS), W_lo, b_lo.reshape(1, LO))
    return out
